# baseline (device time: 26488 ns/iter reference)
import jax
import jax.numpy as jnp
from jax import lax
from jax.experimental import pallas as pl
from jax.experimental.pallas import tpu as pltpu


def kernel(ids, E):
    T = ids.shape[0]
    Vs, D = E.shape

    def body(ids_ref, e_ref, out_ref, acc_ref, recv_ref, send_sem, recv_sem):
        my_x = lax.axis_index("x")
        my_y = lax.axis_index("y")
        nbr = (my_x, 1 - my_y)

        barrier = pltpu.get_barrier_semaphore()
        pl.semaphore_signal(
            barrier, inc=1, device_id=nbr, device_id_type=pl.DeviceIdType.MESH
        )
        pl.semaphore_wait(barrier, 1)

        lo = my_y * Vs

        def gather(i, _):
            idx = ids_ref[i]
            inb = (idx >= lo) & (idx < lo + Vs)
            li = jnp.clip(idx - lo, 0, Vs - 1)
            row = e_ref[pl.ds(li, 1), :]
            acc_ref[pl.ds(i, 1), :] = jnp.where(inb, row, 0.0)
            return 0

        lax.fori_loop(0, T, gather, 0)

        rdma = pltpu.make_async_remote_copy(
            src_ref=acc_ref,
            dst_ref=recv_ref,
            send_sem=send_sem,
            recv_sem=recv_sem,
            device_id=nbr,
            device_id_type=pl.DeviceIdType.MESH,
        )
        rdma.start()
        rdma.wait()

        out_ref[:, :] = acc_ref[:, :] + recv_ref[:, :]

    return pl.pallas_call(
        body,
        out_shape=jax.ShapeDtypeStruct((T, D), jnp.float32),
        in_specs=[
            pl.BlockSpec(memory_space=pltpu.SMEM),
            pl.BlockSpec(memory_space=pltpu.VMEM),
        ],
        out_specs=pl.BlockSpec(memory_space=pltpu.VMEM),
        scratch_shapes=[
            pltpu.VMEM((T, D), jnp.float32),
            pltpu.VMEM((T, D), jnp.float32),
            pltpu.SemaphoreType.DMA,
            pltpu.SemaphoreType.DMA,
        ],
        compiler_params=pltpu.CompilerParams(collective_id=0),
    )(ids, E)


# device time: 16052 ns/iter; 1.6501x vs baseline; 1.6501x over previous
import jax
import jax.numpy as jnp
from jax import lax
from jax.experimental import pallas as pl
from jax.experimental.pallas import tpu as pltpu

TAIL = 96
CS_PIPE = (96, 48, 16)
CS = CS_PIPE + (TAIL, TAIL)
NP = len(CS_PIPE)
C = len(CS)
OFF = [sum(CS[:c]) for c in range(C)]
G_ROWS = sum(CS)


def kernel(ids, E):
    T = ids.shape[0]
    Vs, D = E.shape
    HALF = T // 2
    PIPE = HALF - TAIL
    assert sum(CS_PIPE) == PIPE

    def body(ids_s, e_any, out_ref, e_ref, g_ref, gbf_ref, yrecv_ref,
             xstage_ref, xrecv_ref, esem,
             ysend_sems, yrecv_sems, xsend_sems, xrecv_sems):
        my_x = lax.axis_index("x")
        my_y = lax.axis_index("y")
        ynbr = (my_x, 1 - my_y)
        xnbr = (1 - my_x, my_y)

        ecopy = pltpu.make_async_copy(e_any, e_ref.at[pl.ds(0, Vs)], esem)
        ecopy.start()
        e_ref[pl.ds(Vs, 8), :] = jnp.zeros((8, D), jnp.float32)

        barrier = pltpu.get_barrier_semaphore()
        for nbr in (ynbr, xnbr):
            pl.semaphore_signal(
                barrier, inc=1, device_id=nbr,
                device_id_type=pl.DeviceIdType.MESH,
            )

        lo = my_y * Vs
        half_off = my_x * HALF

        def gather():
            for i in range(G_ROWS):
                if i < PIPE:
                    tok = half_off + i
                elif i < PIPE + TAIL:
                    tok = PIPE + (i - PIPE)
                else:
                    tok = HALF + PIPE + (i - PIPE - TAIL)
                idx = ids_s[tok]
                inb = (idx >= lo) & (idx < lo + Vs)
                li = jnp.where(inb, idx - lo, Vs)
                g_ref[pl.ds(i, 1), :] = e_ref[pl.ds(li, 1), :]

        y_rdmas = []
        x_rdmas = []

        def ysend(c):
            r = pltpu.make_async_remote_copy(
                src_ref=gbf_ref.at[pl.ds(OFF[c], CS[c])],
                dst_ref=yrecv_ref.at[pl.ds(OFF[c], CS[c])],
                send_sem=ysend_sems.at[c],
                recv_sem=yrecv_sems.at[c],
                device_id=ynbr,
                device_id_type=pl.DeviceIdType.MESH,
            )
            r.start()
            y_rdmas.append(r)

        def combine_pipe_xsend(c):
            y_rdmas[c].wait_recv()
            s = (
                g_ref[pl.ds(OFF[c], CS[c]), :]
                + yrecv_ref[pl.ds(OFF[c], CS[c]), :].astype(jnp.float32)
            )
            out_ref[pl.ds(half_off + OFF[c], CS[c]), :] = s
            xstage_ref[pl.ds(OFF[c], CS[c]), :] = s.astype(jnp.bfloat16)
            r = pltpu.make_async_remote_copy(
                src_ref=xstage_ref.at[pl.ds(OFF[c], CS[c])],
                dst_ref=xrecv_ref.at[pl.ds(OFF[c], CS[c])],
                send_sem=xsend_sems.at[c],
                recv_sem=xrecv_sems.at[c],
                device_id=xnbr,
                device_id_type=pl.DeviceIdType.MESH,
            )
            r.start()
            x_rdmas.append(r)

        def combine_tail(c, out_row):
            y_rdmas[c].wait_recv()
            out_ref[pl.ds(out_row, TAIL), :] = (
                g_ref[pl.ds(OFF[c], CS[c]), :]
                + yrecv_ref[pl.ds(OFF[c], CS[c]), :].astype(jnp.float32)
            )

        with jax.named_scope("ecopy_wait"):
            ecopy.wait()
        with jax.named_scope("gather"):
            gather()
        with jax.named_scope("cast_bf16"):
            gbf_ref[:, :] = g_ref[:, :].astype(jnp.bfloat16)
        with jax.named_scope("barrier"):
            pl.semaphore_wait(barrier, 2)
        with jax.named_scope("ysend"):
            for c in range(C):
                ysend(c)
        for c in range(NP):
            with jax.named_scope(f"combine_xsend#c={c}"):
                combine_pipe_xsend(c)
        with jax.named_scope("combine_tail#c=0"):
            combine_tail(NP, PIPE)
        with jax.named_scope("combine_tail#c=1"):
            combine_tail(NP + 1, HALF + PIPE)

        with jax.named_scope("final_waits"):
            other_off = (1 - my_x) * HALF
            for c in range(NP):
                x_rdmas[c].wait_recv()
                out_ref[pl.ds(other_off + OFF[c], CS_PIPE[c]), :] = (
                    xrecv_ref[pl.ds(OFF[c], CS_PIPE[c]), :].astype(jnp.float32)
                )
            for c in range(C):
                y_rdmas[c].wait_send()
            for c in range(NP):
                x_rdmas[c].wait_send()

    return pl.pallas_call(
        body,
        out_shape=jax.ShapeDtypeStruct((T, D), jnp.float32),
        in_specs=[
            pl.BlockSpec(memory_space=pltpu.SMEM),
            pl.BlockSpec(memory_space=pl.ANY),
        ],
        out_specs=pl.BlockSpec(memory_space=pltpu.VMEM),
        scratch_shapes=[
            pltpu.VMEM((Vs + 8, D), jnp.float32),
            pltpu.VMEM((G_ROWS, D), jnp.float32),
            pltpu.VMEM((G_ROWS, D), jnp.bfloat16),
            pltpu.VMEM((G_ROWS, D), jnp.bfloat16),
            pltpu.VMEM((sum(CS_PIPE), D), jnp.bfloat16),
            pltpu.VMEM((sum(CS_PIPE), D), jnp.bfloat16),
            pltpu.SemaphoreType.DMA,
            pltpu.SemaphoreType.DMA((C,)),
            pltpu.SemaphoreType.DMA((C,)),
            pltpu.SemaphoreType.DMA((NP,)),
            pltpu.SemaphoreType.DMA((NP,)),
        ],
        compiler_params=pltpu.CompilerParams(collective_id=0),
    )(ids, E)
